# baseline (speedup 1.0000x reference)
import functools

import jax
import jax.numpy as jnp
from jax import lax
from jax.experimental import pallas as pl
from jax.experimental.pallas import tpu as pltpu
from jax.experimental.pallas import tpu_sc as plsc

_N_WORKERS = 32
_NBUF = 3


def _gather_kernel(n_rows, chunk, d):
    b_per_w = n_rows // _N_WORKERS
    n_chunks = b_per_w // chunk
    mesh = plsc.VectorSubcoreMesh(core_axis_name="c", subcore_axis_name="s")

    @functools.partial(
        pl.kernel,
        mesh=mesh,
        compiler_params=pltpu.CompilerParams(use_tc_tiling_on_sc=False),
        out_type=jax.ShapeDtypeStruct((n_rows, d), jnp.float32),
        scratch_types=[
            pltpu.VMEM((n_chunks, chunk), jnp.int32),
            *[pltpu.VMEM((chunk, d), jnp.float32) for _ in range(_NBUF)],
            *[pltpu.SemaphoreType.DMA for _ in range(2 * _NBUF)],
        ],
    )
    def k(idx_hbm, table_hbm, out_hbm, idx_v, *bufs_and_sems):
        rows = bufs_and_sems[:_NBUF]
        gsems = bufs_and_sems[_NBUF : 2 * _NBUF]
        ssems = bufs_and_sems[2 * _NBUF :]

        cid = lax.axis_index("c")
        sid = lax.axis_index("s")
        wid = sid * 2 + cid
        base = wid * b_per_w

        pltpu.sync_copy(idx_hbm.at[pl.ds(wid * n_chunks, n_chunks)], idx_v)

        g_copies = {}
        s_copies = {}

        def start_gather(c):
            slot = c % _NBUF
            g_copies[c] = pltpu.async_copy(
                table_hbm.at[idx_v.at[c]], rows[slot], gsems[slot]
            )

        def start_store(c):
            slot = c % _NBUF
            s_copies[c] = pltpu.async_copy(
                rows[slot], out_hbm.at[pl.ds(base + c * chunk, chunk)], ssems[slot]
            )

        start_gather(0)
        if n_chunks > 1:
            start_gather(1)
        for g in range(n_chunks):
            g_copies[g].wait()
            start_store(g)
            c = g + 2
            if c < n_chunks:
                if c - _NBUF >= 0:
                    s_copies[c - _NBUF].wait()
                start_gather(c)
        for c in range(max(0, n_chunks - _NBUF), n_chunks):
            s_copies[c].wait()

    return k


def kernel(x, table):
    b, f = x.shape
    v, d = table.shape
    n_rows = b * f
    chunk = 1024
    idx = x.reshape(n_rows // chunk, chunk).astype(jnp.int32)
    out = _gather_kernel(n_rows, chunk, d)(idx, table)
    return out.reshape(b, f, d)


# speedup vs baseline: 2.0135x; 2.0135x over previous
import functools

import jax
import jax.numpy as jnp
from jax import lax
from jax.experimental import pallas as pl
from jax.experimental.pallas import tpu as pltpu
from jax.experimental.pallas import tpu_sc as plsc

_N_WORKERS = 32
_NBUF = 3
_TW = 32768
_BW = 512


def _table_to_rowmajor(table_t, v, d):
    q = _TW // 4
    n_blocks = pl.cdiv(v, _TW)

    def body(in_ref, out_ref):
        x = in_ref[...]
        t = x.T
        out_ref[...] = jnp.concatenate(
            [t[c * q : (c + 1) * q, :] for c in range(4)], axis=1
        )

    return pl.pallas_call(
        body,
        grid=(n_blocks,),
        in_specs=[pl.BlockSpec((d, _TW), lambda i: (0, i))],
        out_specs=pl.BlockSpec((q, 4 * d), lambda i: (i, 0)),
        out_shape=jax.ShapeDtypeStruct((n_blocks * q, 4 * d), jnp.float32),
    )(table_t)


def _out_to_native(out_packed, b, f, d):
    q = _BW // 4
    n_blocks = b // _BW
    k_sub = min(16, n_blocks)

    def body(in_ref, out_ref):
        x = in_ref[0]
        eye = jnp.eye(4 * d, dtype=jnp.float32)
        dn = (((0,), (0,)), ((), ()))
        pieces = []
        for k in range(k_sub):
            t = lax.dot_general(
                x[k * q : (k + 1) * q, :],
                eye,
                dn,
                preferred_element_type=jnp.float32,
            )
            pieces.extend(t[c * d : (c + 1) * d, :] for c in range(4))
        out_ref[0] = jnp.concatenate(pieces, axis=1)

    return pl.pallas_call(
        body,
        grid=(f, n_blocks // k_sub),
        in_specs=[pl.BlockSpec((1, k_sub * q, 4 * d), lambda j, i: (j, i, 0))],
        out_specs=pl.BlockSpec((1, d, k_sub * _BW), lambda j, i: (j, 0, i)),
        out_shape=jax.ShapeDtypeStruct((f, d, b), jnp.float32),
    )(out_packed)


def _sc_gather(n_rows, chunk, d):
    b_per_w = n_rows // _N_WORKERS
    n_chunks = b_per_w // chunk
    mesh = plsc.VectorSubcoreMesh(core_axis_name="c", subcore_axis_name="s")

    @functools.partial(
        pl.kernel,
        mesh=mesh,
        compiler_params=pltpu.CompilerParams(use_tc_tiling_on_sc=False),
        out_type=jax.ShapeDtypeStruct((n_rows, d), jnp.float32),
        scratch_types=[
            pltpu.VMEM((n_chunks, chunk), jnp.int32),
            *[pltpu.VMEM((chunk, d), jnp.float32) for _ in range(_NBUF)],
            *[pltpu.SemaphoreType.DMA for _ in range(2 * _NBUF)],
        ],
    )
    def k(idx_hbm, table_hbm, out_hbm, idx_v, *bufs_and_sems):
        rows = bufs_and_sems[:_NBUF]
        gsems = bufs_and_sems[_NBUF : 2 * _NBUF]
        ssems = bufs_and_sems[2 * _NBUF :]

        cid = lax.axis_index("c")
        sid = lax.axis_index("s")
        wid = sid * 2 + cid
        base = wid * b_per_w

        pltpu.sync_copy(idx_hbm.at[pl.ds(wid * n_chunks, n_chunks)], idx_v)

        g_copies = {}
        s_copies = {}

        def start_gather(c):
            slot = c % _NBUF
            g_copies[c] = pltpu.async_copy(
                table_hbm.at[idx_v.at[c]], rows[slot], gsems[slot]
            )

        def start_store(c):
            slot = c % _NBUF
            s_copies[c] = pltpu.async_copy(
                rows[slot], out_hbm.at[pl.ds(base + c * chunk, chunk)], ssems[slot]
            )

        start_gather(0)
        if n_chunks > 1:
            start_gather(1)
        for g in range(n_chunks):
            g_copies[g].wait()
            start_store(g)
            c = g + 2
            if c < n_chunks:
                if c - _NBUF >= 0:
                    s_copies[c - _NBUF].wait()
                start_gather(c)
        for c in range(max(0, n_chunks - _NBUF), n_chunks):
            s_copies[c].wait()

    return k


def _prep_idx(x, b, f, n_rows):
    s = jnp.arange(_BW, dtype=jnp.int32)
    perm = (s % 4) * (_BW // 4) + s // 4
    xq = jnp.take(x.T.reshape(f, b // _BW, _BW), perm, axis=2)
    xv = xq.astype(jnp.int32).reshape(n_rows)
    tq = _TW // 4
    i_blk = xv // _TW
    rem = xv % _TW
    return i_blk * _TW + (rem % tq) * 4 + rem // tq


def kernel(x, table):
    b, f = x.shape
    v, d = table.shape
    n_rows = b * f
    chunk = 1024

    table_rm = _table_to_rowmajor(table.T, v, d)
    v_pad = 4 * table_rm.shape[0]

    idx = _prep_idx(x, b, f, n_rows).reshape(n_rows // chunk, chunk)

    out_rows = _sc_gather(n_rows, chunk, d)(idx, table_rm.reshape(v_pad, d))

    out_packed = out_rows.reshape(f, b * d // 128, 128)
    out_phys = _out_to_native(out_packed, b, f, d)
    return jnp.transpose(out_phys, (2, 0, 1))


# speedup vs baseline: 3.3666x; 1.6720x over previous
import functools

import jax
import jax.numpy as jnp
from jax import lax
from jax.experimental import pallas as pl
from jax.experimental.pallas import tpu as pltpu
from jax.experimental.pallas import tpu_sc as plsc

_N_WORKERS = 32
_NBUF = 3
_TW = 32768
_BW = 512


def _table_to_rowmajor(table_t, v, d):
    q = _TW // 4
    n_blocks = pl.cdiv(v, _TW)

    def body(in_ref, out_ref):
        x = in_ref[...]
        a = jnp.concatenate([x[:, c * q : (c + 1) * q] for c in range(4)], axis=0)
        eye = jnp.eye(4 * d, dtype=jnp.float32)
        dn = (((0,), (0,)), ((), ()))
        out_ref[...] = lax.dot_general(a, eye, dn, preferred_element_type=jnp.float32)

    return pl.pallas_call(
        body,
        grid=(n_blocks,),
        in_specs=[pl.BlockSpec((d, _TW), lambda i: (0, i))],
        out_specs=pl.BlockSpec((q, 4 * d), lambda i: (i, 0)),
        out_shape=jax.ShapeDtypeStruct((n_blocks * q, 4 * d), jnp.float32),
    )(table_t)


def _out_to_native(out_packed, b, f, d):
    q = _BW // 4
    n_blocks = b // _BW
    k_sub = min(16, n_blocks)

    def body(in_ref, out_ref):
        x = in_ref[0]
        eye = jnp.eye(4 * d, dtype=jnp.float32)
        dn = (((0,), (0,)), ((), ()))
        pieces = []
        for k in range(k_sub):
            t = lax.dot_general(
                x[k * q : (k + 1) * q, :],
                eye,
                dn,
                preferred_element_type=jnp.float32,
            )
            pieces.extend(t[c * d : (c + 1) * d, :] for c in range(4))
        out_ref[0] = jnp.concatenate(pieces, axis=1)

    return pl.pallas_call(
        body,
        grid=(f, n_blocks // k_sub),
        in_specs=[pl.BlockSpec((1, k_sub * q, 4 * d), lambda j, i: (j, i, 0))],
        out_specs=pl.BlockSpec((1, d, k_sub * _BW), lambda j, i: (j, 0, i)),
        out_shape=jax.ShapeDtypeStruct((f, d, b), jnp.float32),
    )(out_packed)


def _sc_gather(n_rows, chunk, d):
    b_per_w = n_rows // _N_WORKERS
    n_chunks = b_per_w // chunk
    mesh = plsc.VectorSubcoreMesh(core_axis_name="c", subcore_axis_name="s")

    @functools.partial(
        pl.kernel,
        mesh=mesh,
        compiler_params=pltpu.CompilerParams(use_tc_tiling_on_sc=False),
        out_type=jax.ShapeDtypeStruct((n_rows, d), jnp.float32),
        scratch_types=[
            pltpu.VMEM((n_chunks, chunk), jnp.int32),
            *[pltpu.VMEM((chunk, d), jnp.float32) for _ in range(_NBUF)],
            *[pltpu.SemaphoreType.DMA for _ in range(2 * _NBUF)],
        ],
    )
    def k(idx_hbm, table_hbm, out_hbm, idx_v, *bufs_and_sems):
        rows = bufs_and_sems[:_NBUF]
        gsems = bufs_and_sems[_NBUF : 2 * _NBUF]
        ssems = bufs_and_sems[2 * _NBUF :]

        cid = lax.axis_index("c")
        sid = lax.axis_index("s")
        wid = sid * 2 + cid
        base = wid * b_per_w

        pltpu.sync_copy(idx_hbm.at[pl.ds(wid * n_chunks, n_chunks)], idx_v)

        g_copies = {}
        s_copies = {}

        def start_gather(c):
            slot = c % _NBUF
            g_copies[c] = pltpu.async_copy(
                table_hbm.at[idx_v.at[c]], rows[slot], gsems[slot]
            )

        def start_store(c):
            slot = c % _NBUF
            s_copies[c] = pltpu.async_copy(
                rows[slot], out_hbm.at[pl.ds(base + c * chunk, chunk)], ssems[slot]
            )

        start_gather(0)
        if n_chunks > 1:
            start_gather(1)
        for g in range(n_chunks):
            g_copies[g].wait()
            start_store(g)
            c = g + 2
            if c < n_chunks:
                if c - _NBUF >= 0:
                    s_copies[c - _NBUF].wait()
                start_gather(c)
        for c in range(max(0, n_chunks - _NBUF), n_chunks):
            s_copies[c].wait()

    return k


def _prep_idx(x, b, f, n_rows):
    s = jnp.arange(_BW, dtype=jnp.int32)
    perm = (s % 4) * (_BW // 4) + s // 4
    xq = jnp.take(x.T.reshape(f, b // _BW, _BW), perm, axis=2)
    xv = xq.astype(jnp.int32).reshape(n_rows)
    tq = _TW // 4
    i_blk = xv // _TW
    rem = xv % _TW
    return i_blk * _TW + (rem % tq) * 4 + rem // tq


def kernel(x, table):
    b, f = x.shape
    v, d = table.shape
    n_rows = b * f
    chunk = 1024

    table_rm = _table_to_rowmajor(table.T, v, d)
    v_pad = 4 * table_rm.shape[0]

    idx = _prep_idx(x, b, f, n_rows).reshape(n_rows // chunk, chunk)

    out_rows = _sc_gather(n_rows, chunk, d)(idx, table_rm.reshape(v_pad, d))

    out_packed = out_rows.reshape(f, b * d // 128, 128)
    out_phys = _out_to_native(out_packed, b, f, d)
    return jnp.transpose(out_phys, (2, 0, 1))


# speedup vs baseline: 3.6260x; 1.0771x over previous
import functools

import jax
import jax.numpy as jnp
from jax import lax
from jax.experimental import pallas as pl
from jax.experimental.pallas import tpu as pltpu
from jax.experimental.pallas import tpu_sc as plsc

_N_WORKERS = 32
_NBUF = 3
_TW = 32768
_BW = 512


def _table_to_rowmajor(table_t, v, d):
    q = _TW // 4
    n_blocks = pl.cdiv(v, _TW)

    def body(in_ref, out_ref):
        x = in_ref[...]
        a = jnp.concatenate([x[:, c * q : (c + 1) * q] for c in range(4)], axis=0)
        eye = jnp.eye(4 * d, dtype=jnp.float32)
        dn = (((0,), (0,)), ((), ()))
        out_ref[...] = lax.dot_general(a, eye, dn, preferred_element_type=jnp.float32)

    return pl.pallas_call(
        body,
        grid=(n_blocks,),
        in_specs=[pl.BlockSpec((d, _TW), lambda i: (0, i))],
        out_specs=pl.BlockSpec((q, 4 * d), lambda i: (i, 0)),
        out_shape=jax.ShapeDtypeStruct((n_blocks * q, 4 * d), jnp.float32),
    )(table_t)


def _out_to_native(out_packed, b, f, d):
    q = _BW // 4
    n_blocks = b // _BW
    k_sub = min(32, n_blocks)

    def body(in_ref, out_ref):
        x = in_ref[0]
        eye = jnp.eye(4 * d, dtype=jnp.float32)
        dn = (((0,), (0,)), ((), ()))
        pieces = []
        for k in range(k_sub):
            t = lax.dot_general(
                x[k * q : (k + 1) * q, :],
                eye,
                dn,
                preferred_element_type=jnp.float32,
            )
            pieces.extend(t[c * d : (c + 1) * d, :] for c in range(4))
        out_ref[0] = jnp.concatenate(pieces, axis=1)

    return pl.pallas_call(
        body,
        grid=(f, n_blocks // k_sub),
        in_specs=[pl.BlockSpec((1, k_sub * q, 4 * d), lambda j, i: (j, i, 0))],
        out_specs=pl.BlockSpec((1, d, k_sub * _BW), lambda j, i: (j, 0, i)),
        out_shape=jax.ShapeDtypeStruct((f, d, b), jnp.float32),
    )(out_packed)


def _sc_gather(n_rows, chunk, d):
    b_per_w = n_rows // _N_WORKERS
    n_chunks = b_per_w // chunk
    mesh = plsc.VectorSubcoreMesh(core_axis_name="c", subcore_axis_name="s")

    @functools.partial(
        pl.kernel,
        mesh=mesh,
        compiler_params=pltpu.CompilerParams(use_tc_tiling_on_sc=False),
        out_type=jax.ShapeDtypeStruct((n_rows, d), jnp.float32),
        scratch_types=[
            pltpu.VMEM((n_chunks, chunk), jnp.int32),
            *[pltpu.VMEM((chunk, d), jnp.float32) for _ in range(_NBUF)],
            *[pltpu.SemaphoreType.DMA for _ in range(2 * _NBUF)],
        ],
    )
    def k(idx_hbm, table_hbm, out_hbm, idx_v, *bufs_and_sems):
        rows = bufs_and_sems[:_NBUF]
        gsems = bufs_and_sems[_NBUF : 2 * _NBUF]
        ssems = bufs_and_sems[2 * _NBUF :]

        cid = lax.axis_index("c")
        sid = lax.axis_index("s")
        wid = sid * 2 + cid
        base = wid * b_per_w

        pltpu.sync_copy(idx_hbm.at[pl.ds(wid * n_chunks, n_chunks)], idx_v)

        g_copies = {}
        s_copies = {}

        def start_gather(c):
            slot = c % _NBUF
            g_copies[c] = pltpu.async_copy(
                table_hbm.at[idx_v.at[c]], rows[slot], gsems[slot]
            )

        def start_store(c):
            slot = c % _NBUF
            s_copies[c] = pltpu.async_copy(
                rows[slot], out_hbm.at[pl.ds(base + c * chunk, chunk)], ssems[slot]
            )

        start_gather(0)
        if n_chunks > 1:
            start_gather(1)
        for g in range(n_chunks):
            g_copies[g].wait()
            start_store(g)
            c = g + 2
            if c < n_chunks:
                if c - _NBUF >= 0:
                    s_copies[c - _NBUF].wait()
                start_gather(c)
        for c in range(max(0, n_chunks - _NBUF), n_chunks):
            s_copies[c].wait()

    return k


def _prep_idx(x, b, f, n_rows):
    s = jnp.arange(_BW, dtype=jnp.int32)
    perm = (s % 4) * (_BW // 4) + s // 4
    xq = jnp.take(x.T.reshape(f, b // _BW, _BW), perm, axis=2)
    xv = xq.astype(jnp.int32).reshape(n_rows)
    tq = _TW // 4
    i_blk = xv // _TW
    rem = xv % _TW
    return i_blk * _TW + (rem % tq) * 4 + rem // tq


def kernel(x, table):
    b, f = x.shape
    v, d = table.shape
    n_rows = b * f
    chunk = 1024

    table_rm = _table_to_rowmajor(table.T, v, d)
    v_pad = 4 * table_rm.shape[0]

    idx = _prep_idx(x, b, f, n_rows).reshape(n_rows // chunk, chunk)

    out_rows = _sc_gather(n_rows, chunk, d)(idx, table_rm.reshape(v_pad, d))

    out_packed = out_rows.reshape(f, b * d // 128, 128)
    out_phys = _out_to_native(out_packed, b, f, d)
    return jnp.transpose(out_phys, (2, 0, 1))
